# baseline (speedup 1.0000x reference)
import dataclasses
import functools

import jax
import jax.numpy as jnp
from jax import lax
from jax.experimental import pallas as pl
from jax.experimental.pallas import tpu as pltpu
from jax.experimental.pallas import tpu_sc as plsc

_N = 10000
_E = 320000
_DIN = 128
_DH = 128
_DOUT = 64
_G = 64

_NC = 2
_NS = 16
_NW = _NC * _NS
_NPAD = 10240
_CHUNK = 80
_EPAD = 4096 * _CHUNK
_ROWS_T = _NPAD // _NS
_DINV_T = _NPAD // _NS
_BLKC = 16

_L1_NCH0, _L1_NCH1 = 240, 16
_L2_NCH0, _L2_NCH1 = 240, 16
_DCHUNK = _EPAD // _CHUNK // _NS

_mesh = plsc.VectorSubcoreMesh(core_axis_name="c", subcore_axis_name="s")

_sc_params = pltpu.CompilerParams()
if "needs_layout_passes" in pltpu.CompilerParams.__dataclass_fields__:
    _sc_params = dataclasses.replace(_sc_params, needs_layout_passes=False)


def _zero16(ref, base):
    ref[pl.ds(base, 16)] = jnp.zeros((16,), jnp.float32)


def _newton_rsqrt(d):
    half = jnp.full((16,), 0.5, jnp.float32)
    three_half = jnp.full((16,), 1.5, jnp.float32)
    magic = jnp.full((16,), 0x5F3759DF, jnp.int32)
    one = jnp.full((16,), 1, jnp.int32)
    y = plsc.bitcast(magic - lax.shift_right_arithmetic(plsc.bitcast(d, jnp.int32), one),
                     jnp.float32)
    for _ in range(3):
        y = y * (three_half - half * d * y * y)
    return y


def _scale_rows(rows, norm_v, width):
    @pl.loop(0, _CHUNK)
    def _(e):
        spl = plsc.load_gather(norm_v, [jnp.zeros((16,), jnp.int32) + e])
        for j in range(width // 16):
            sl = pl.ds(16 * j, 16)
            rows[e, sl] = rows[e, sl] * spl


def _prop_blocks(m_hbm, src_hbm, dst_hbm, w_hbm, base_row, nb,
                 sidxb, didxb, wb, norm_v, rows0, rows1, dinv_v, out_sh,
                 semi, sem0, sem1):
    def idx_issue(b, p):
        base = base_row + b * _BLKC
        return [
            pltpu.async_copy(src_hbm.at[pl.ds(base, _BLKC)], sidxb[p], semi),
            pltpu.async_copy(dst_hbm.at[pl.ds(base, _BLKC)], didxb[p], semi),
            pltpu.async_copy(w_hbm.at[pl.ds(base, _BLKC)], wb[p], semi),
        ]

    def idx_drain(p):
        pltpu.make_async_copy(src_hbm.at[pl.ds(0, _BLKC)], sidxb[p], semi).wait()
        pltpu.make_async_copy(dst_hbm.at[pl.ds(0, _BLKC)], didxb[p], semi).wait()
        pltpu.make_async_copy(w_hbm.at[pl.ds(0, _BLKC)], wb[p], semi).wait()

    def gather_issue(p, j, buf, sem):
        pltpu.async_copy(m_hbm.at[sidxb[p].at[j]], buf, sem)

    def process(p, j, buf, sem):
        for kk in range(_CHUNK // 16):
            sl = pl.ds(kk * 16, 16)
            ds_ = plsc.load_gather(dinv_v, [sidxb[p][j, sl]])
            dd_ = plsc.load_gather(dinv_v, [didxb[p][j, sl]])
            norm_v[sl] = ds_ * wb[p][j, sl] * dd_
        pltpu.make_async_copy(m_hbm.at[pl.ds(0, _CHUNK)], buf, sem).wait()
        _scale_rows(buf, norm_v, _DH)
        pltpu.sync_copy(buf, out_sh.at[didxb[p].at[j]], add=True)

    for h in idx_issue(0, 0):
        h.wait()

    def block(b, p):
        @pl.when(b < nb - 1)
        def _():
            idx_issue(b + 1, 1 - p)

        gather_issue(p, 0, rows0, sem0)

        @pl.loop(0, _BLKC // 2)
        def _(q):
            j0 = 2 * q
            gather_issue(p, j0 + 1, rows1, sem1)
            process(p, j0, rows0, sem0)

            @pl.when(q < _BLKC // 2 - 1)
            def _():
                gather_issue(p, j0 + 2, rows0, sem0)

            process(p, j0 + 1, rows1, sem1)

        @pl.when(b < nb - 1)
        def _():
            idx_drain(1 - p)

    @pl.loop(0, nb // 2)
    def _(bb):
        block(2 * bb, 0)
        block(2 * bb + 1, 1)

    @pl.when(lax.rem(nb, 2) == 1)
    def _():
        block(nb - 1, 0)


def _sc_layer1(m1, src80, dst80, w80):
    @functools.partial(
        pl.kernel,
        out_type=[
            jax.ShapeDtypeStruct((_NC, _NPAD, _DH), jnp.float32),
            jax.ShapeDtypeStruct((_NPAD,), jnp.float32),
        ],
        mesh=_mesh,
        compiler_params=_sc_params,
        scratch_types=[
            pltpu.VMEM((_BLKC, _CHUNK), jnp.int32),
            pltpu.VMEM((_BLKC, _CHUNK), jnp.int32),
            pltpu.VMEM((_BLKC, _CHUNK), jnp.int32),
            pltpu.VMEM((_BLKC, _CHUNK), jnp.int32),
            pltpu.VMEM((_BLKC, _CHUNK), jnp.float32),
            pltpu.VMEM((_BLKC, _CHUNK), jnp.float32),
            pltpu.VMEM((_CHUNK,), jnp.float32),
            pltpu.VMEM((_CHUNK, _DH), jnp.float32),
            pltpu.VMEM((_CHUNK, _DH), jnp.float32),
            pltpu.VMEM((_NPAD,), jnp.float32),
            pltpu.VMEM((_DINV_T,), jnp.float32),
            pltpu.VMEM_SHARED((_NPAD,), jnp.float32),
            pltpu.VMEM_SHARED((_NPAD,), jnp.float32),
            pltpu.VMEM_SHARED((_NPAD, _DH), jnp.float32),
            pltpu.SemaphoreType.DMA,
            pltpu.SemaphoreType.DMA,
            pltpu.SemaphoreType.DMA,
            pltpu.SemaphoreType.DMA,
        ],
    )
    def k(m1_hbm, src_hbm, dst_hbm, w_hbm, p1_hbm, dinv_hbm,
          sidxb0, sidxb1, didxb0, didxb1, wb0, wb1, norm_v, rows0, rows1,
          dinv_v, tmp_v, deg_sh, dinv_sh, out_sh, semi, sem0, sem1, semd):
        cid = lax.axis_index("c")
        sid = lax.axis_index("s")
        sidxb, didxb, wb = [sidxb0, sidxb1], [didxb0, didxb1], [wb0, wb1]

        @pl.loop(0, _CHUNK)
        def _(r):
            for j in range(_DH // 16):
                rows0[r, pl.ds(16 * j, 16)] = jnp.zeros((16,), jnp.float32)

        @pl.loop(0, _DINV_T // 16)
        def _(i):
            _zero16(tmp_v, i * 16)

        pltpu.sync_copy(tmp_v, deg_sh.at[pl.ds(sid * _DINV_T, _DINV_T)])
        for kk in range(_ROWS_T // _CHUNK):
            pltpu.sync_copy(
                rows0, out_sh.at[pl.ds(sid * _ROWS_T + _CHUNK * kk, _CHUNK)])
        plsc.subcore_barrier()

        nbd = _DCHUNK // _BLKC

        def deg_issue(g, p):
            base = sid * _DCHUNK + g * _BLKC
            return [
                pltpu.async_copy(dst_hbm.at[pl.ds(base, _BLKC)], didxb[p],
                                 semi),
                pltpu.async_copy(w_hbm.at[pl.ds(base, _BLKC)], wb[p], semi),
            ]

        def deg_drain(p):
            pltpu.make_async_copy(dst_hbm.at[pl.ds(0, _BLKC)], didxb[p],
                                  semi).wait()
            pltpu.make_async_copy(w_hbm.at[pl.ds(0, _BLKC)], wb[p],
                                  semi).wait()

        for h in deg_issue(0, 0):
            h.wait()

        def deg_block(g, p):
            @pl.when(g < nbd - 1)
            def _():
                deg_issue(g + 1, 1 - p)

            hs = []
            for j in range(_BLKC):
                hs.append(pltpu.async_copy(
                    wb[p].at[j], deg_sh.at[didxb[p].at[j]], semd, add=True))
            for h in hs:
                h.wait()

            @pl.when(g < nbd - 1)
            def _():
                deg_drain(1 - p)

        @pl.loop(0, nbd // 2)
        def _(gg):
            deg_block(2 * gg, 0)
            deg_block(2 * gg + 1, 1)

        pl.delay(10000)
        plsc.subcore_barrier()

        nbase = sid * _DINV_T
        pltpu.sync_copy(deg_sh.at[pl.ds(nbase, _DINV_T)], tmp_v)

        @pl.loop(0, _DINV_T // 16)
        def _(i):
            d = tmp_v[pl.ds(i * 16, 16)] + jnp.full((16,), 1.0, jnp.float32)
            tmp_v[pl.ds(i * 16, 16)] = _newton_rsqrt(d)

        pltpu.sync_copy(tmp_v, dinv_sh.at[pl.ds(nbase, _DINV_T)])

        @pl.when(cid == 0)
        def _():
            pltpu.sync_copy(tmp_v, dinv_hbm.at[pl.ds(nbase, _DINV_T)])

        plsc.subcore_barrier()

        pltpu.sync_copy(dinv_sh, dinv_v)
        base_row = jnp.where(cid == 0, sid * _L1_NCH0,
                             _NS * _L1_NCH0 + sid * _L1_NCH1)
        nb = jnp.where(cid == 0, _L1_NCH0 // _BLKC, _L1_NCH1 // _BLKC)
        _prop_blocks(m1_hbm, src_hbm, dst_hbm, w_hbm, base_row, nb,
                     sidxb, didxb, wb, norm_v, rows0, rows1, dinv_v,
                     out_sh, semi, sem0, sem1)
        plsc.subcore_barrier()

        pltpu.sync_copy(out_sh.at[pl.ds(sid * _ROWS_T, _ROWS_T)],
                        p1_hbm.at[cid, pl.ds(sid * _ROWS_T, _ROWS_T)])

    return k(m1, src80, dst80, w80)


def _sc_layer2(m2, dinv_full, src80, dst80, w80):
    @functools.partial(
        pl.kernel,
        out_type=jax.ShapeDtypeStruct((_NC, _NPAD, _DH), jnp.float32),
        mesh=_mesh,
        compiler_params=_sc_params,
        scratch_types=[
            pltpu.VMEM((_BLKC, _CHUNK), jnp.int32),
            pltpu.VMEM((_BLKC, _CHUNK), jnp.int32),
            pltpu.VMEM((_BLKC, _CHUNK), jnp.int32),
            pltpu.VMEM((_BLKC, _CHUNK), jnp.int32),
            pltpu.VMEM((_BLKC, _CHUNK), jnp.float32),
            pltpu.VMEM((_BLKC, _CHUNK), jnp.float32),
            pltpu.VMEM((_CHUNK,), jnp.float32),
            pltpu.VMEM((_CHUNK, _DH), jnp.float32),
            pltpu.VMEM((_CHUNK, _DH), jnp.float32),
            pltpu.VMEM((_NPAD,), jnp.float32),
            pltpu.VMEM_SHARED((_NPAD, _DH), jnp.float32),
            pltpu.SemaphoreType.DMA,
            pltpu.SemaphoreType.DMA,
            pltpu.SemaphoreType.DMA,
        ],
    )
    def k(m2_hbm, dinv_hbm, src_hbm, dst_hbm, w_hbm, p2_hbm,
          sidxb0, sidxb1, didxb0, didxb1, wb0, wb1, norm_v, rows0, rows1,
          dinv_v, out_sh, semi, sem0, sem1):
        cid = lax.axis_index("c")
        sid = lax.axis_index("s")
        sidxb, didxb, wb = [sidxb0, sidxb1], [didxb0, didxb1], [wb0, wb1]

        hdinv = pltpu.async_copy(dinv_hbm, dinv_v, sem0)

        @pl.loop(0, _CHUNK)
        def _(r):
            for j in range(_DH // 16):
                rows0[r, pl.ds(16 * j, 16)] = jnp.zeros((16,), jnp.float32)

        for kk in range(_ROWS_T // _CHUNK):
            pltpu.sync_copy(
                rows0, out_sh.at[pl.ds(sid * _ROWS_T + _CHUNK * kk, _CHUNK)])
        hdinv.wait()
        plsc.subcore_barrier()

        base_row = jnp.where(cid == 0, sid * _L2_NCH0,
                             _NS * _L2_NCH0 + sid * _L2_NCH1)
        nb = jnp.where(cid == 0, _L2_NCH0 // _BLKC, _L2_NCH1 // _BLKC)
        _prop_blocks(m2_hbm, src_hbm, dst_hbm, w_hbm, base_row, nb,
                     sidxb, didxb, wb, norm_v, rows0, rows1, dinv_v,
                     out_sh, semi, sem0, sem1)
        plsc.subcore_barrier()

        pltpu.sync_copy(out_sh.at[pl.ds(sid * _ROWS_T, _ROWS_T)],
                        p2_hbm.at[cid, pl.ds(sid * _ROWS_T, _ROWS_T)])

    return k(m2, dinv_full, src80, dst80, w80)


_BLK = 1000


def _mm1_body(x_ref, w_ref, o_ref):
    o_ref[...] = jnp.dot(x_ref[...], w_ref[...],
                         preferred_element_type=jnp.float32)


def _mm1(x, W1):
    return pl.pallas_call(
        _mm1_body,
        grid=(_N // _BLK,),
        in_specs=[
            pl.BlockSpec((_BLK, _DIN), lambda i: (i, 0)),
            pl.BlockSpec((_DIN, _DH), lambda i: (0, 0)),
        ],
        out_specs=pl.BlockSpec((_BLK, _DH), lambda i: (i, 0)),
        out_shape=jax.ShapeDtypeStruct((_N, _DH), jnp.float32),
    )(x, W1)


def _mid_body(p1a, p1b, m1, dinv, b1, lw, lb, o_ref):
    dv = dinv[...]
    t = p1a[...] + p1b[...] + dv * dv * m1[...] + b1[...]
    t = jnp.maximum(t, 0.0)
    mu = jnp.mean(t, axis=-1, keepdims=True)
    var = jnp.mean((t - mu) ** 2, axis=-1, keepdims=True)
    o_ref[...] = (t - mu) * lax.rsqrt(var + 1e-5) * lw[...] + lb[...]


def _mid(p1a, p1b, m1, dinv, b1, lw, lb):
    row = lambda i: (i, 0)
    fixed = lambda i: (0, 0)
    return pl.pallas_call(
        _mid_body,
        grid=(_N // _BLK,),
        in_specs=[
            pl.BlockSpec((_BLK, _DH), row),
            pl.BlockSpec((_BLK, _DH), row),
            pl.BlockSpec((_BLK, _DH), row),
            pl.BlockSpec((_BLK, 1), row),
            pl.BlockSpec((1, _DH), fixed),
            pl.BlockSpec((1, _DH), fixed),
            pl.BlockSpec((1, _DH), fixed),
        ],
        out_specs=pl.BlockSpec((_BLK, _DH), row),
        out_shape=jax.ShapeDtypeStruct((_N, _DH), jnp.float32),
    )(p1a, p1b, m1, dinv, b1, lw, lb)


def _fin_body(q2a, q2b, hln, dinv, w2, b2, lw, lb, seg, o_ref, acc):
    i = pl.program_id(0)

    @pl.when(i == 0)
    def _():
        acc[...] = jnp.zeros_like(acc)

    dv = dinv[...]
    t0 = q2a[...] + q2b[...] + dv * dv * hln[...]
    t = jnp.dot(t0, w2[...], preferred_element_type=jnp.float32) + b2[...]
    t = jnp.maximum(t, 0.0)
    mu = jnp.mean(t, axis=-1, keepdims=True)
    var = jnp.mean((t - mu) ** 2, axis=-1, keepdims=True)
    t = (t - mu) * lax.rsqrt(var + 1e-5) * lw[...] + lb[...]

    onehot = (seg[...] == lax.broadcasted_iota(jnp.int32, (1, _G), 1)
              ).astype(jnp.float32)
    t_ext = jnp.concatenate(
        [t, jnp.ones((_BLK, 1), jnp.float32)], axis=1)
    acc[...] += lax.dot_general(onehot, t_ext, (((0,), (0,)), ((), ())),
                                preferred_element_type=jnp.float32)

    @pl.when(i == _N // _BLK - 1)
    def _():
        sums = acc[:, :_DOUT]
        cnt = acc[:, _DOUT:_DOUT + 1]
        pooled = sums / jnp.maximum(cnt, 1.0)
        o_ref[...] = 1.0 / (1.0 + jnp.exp(-pooled))


def _fin(q2a, q2b, hln, dinv, W2, b2, lw, lb, seg):
    row = lambda i: (i, 0)
    fixed = lambda i: (0, 0)
    return pl.pallas_call(
        _fin_body,
        grid=(_N // _BLK,),
        in_specs=[
            pl.BlockSpec((_BLK, _DH), row),
            pl.BlockSpec((_BLK, _DH), row),
            pl.BlockSpec((_BLK, _DH), row),
            pl.BlockSpec((_BLK, 1), row),
            pl.BlockSpec((_DH, _DOUT), fixed),
            pl.BlockSpec((1, _DOUT), fixed),
            pl.BlockSpec((1, _DOUT), fixed),
            pl.BlockSpec((1, _DOUT), fixed),
            pl.BlockSpec((_BLK, 1), row),
        ],
        out_specs=pl.BlockSpec((_G, _G), fixed),
        out_shape=jax.ShapeDtypeStruct((_G, _G), jnp.float32),
        scratch_shapes=[pltpu.VMEM((_G, _DOUT + 1), jnp.float32)],
    )(q2a, q2b, hln, dinv, W2, b2, lw, lb, seg)


def kernel(x, edge_index, edge_weight, data, W1, b1, ln1_w, ln1_b,
           W2, b2, ln2_w, ln2_b):
    epad = _EPAD - _E
    src = jnp.concatenate(
        [edge_index[0], jnp.zeros((epad,), jnp.int32)]
    ).reshape(_EPAD // _CHUNK, _CHUNK)
    dst = jnp.concatenate(
        [edge_index[1], jnp.zeros((epad,), jnp.int32)]
    ).reshape(_EPAD // _CHUNK, _CHUNK)
    ew = jnp.concatenate(
        [edge_weight, jnp.zeros((epad,), jnp.float32)]
    ).reshape(_EPAD // _CHUNK, _CHUNK)

    m1 = _mm1(x, W1)
    p1, dinv_full = _sc_layer1(m1, src, dst, ew)
    dinv = dinv_full[:_N].reshape(_N, 1)

    hln = _mid(p1[0, :_N], p1[1, :_N], m1, dinv,
               b1.reshape(1, _DH), ln1_w.reshape(1, _DH),
               ln1_b.reshape(1, _DH))

    p2 = _sc_layer2(hln, dinv_full, src, dst, ew)

    return _fin(p2[0, :_N], p2[1, :_N], hln, dinv, W2,
                b2.reshape(1, _DOUT), ln2_w.reshape(1, _DOUT),
                ln2_b.reshape(1, _DOUT), data.reshape(_N, 1))


# speedup vs baseline: 1.0490x; 1.0490x over previous
import dataclasses
import functools

import jax
import jax.numpy as jnp
from jax import lax
from jax.experimental import pallas as pl
from jax.experimental.pallas import tpu as pltpu
from jax.experimental.pallas import tpu_sc as plsc

_N = 10000
_E = 320000
_DIN = 128
_DH = 128
_DOUT = 64
_G = 64

_NC = 2
_NS = 16
_NW = _NC * _NS
_NPAD = 10240
_CHUNK = 80
_EPAD = 4096 * _CHUNK
_ROWS_T = _NPAD // _NS
_DINV_T = _NPAD // _NS
_BLKC = 16

_L1_NCH0, _L1_NCH1 = 224, 32
_L2_NCH0, _L2_NCH1 = 224, 32
_DCHUNK = _EPAD // _CHUNK // _NS

_mesh = plsc.VectorSubcoreMesh(core_axis_name="c", subcore_axis_name="s")

_sc_params = pltpu.CompilerParams()
if "needs_layout_passes" in pltpu.CompilerParams.__dataclass_fields__:
    _sc_params = dataclasses.replace(_sc_params, needs_layout_passes=False)


def _zero16(ref, base):
    ref[pl.ds(base, 16)] = jnp.zeros((16,), jnp.float32)


def _newton_rsqrt(d):
    half = jnp.full((16,), 0.5, jnp.float32)
    three_half = jnp.full((16,), 1.5, jnp.float32)
    magic = jnp.full((16,), 0x5F3759DF, jnp.int32)
    one = jnp.full((16,), 1, jnp.int32)
    y = plsc.bitcast(magic - lax.shift_right_arithmetic(plsc.bitcast(d, jnp.int32), one),
                     jnp.float32)
    for _ in range(3):
        y = y * (three_half - half * d * y * y)
    return y


def _scale_rows(rows, norm_v, width):
    @pl.loop(0, _CHUNK)
    def _(e):
        spl = plsc.load_gather(norm_v, [jnp.zeros((16,), jnp.int32) + e])
        for j in range(width // 16):
            sl = pl.ds(16 * j, 16)
            rows[e, sl] = rows[e, sl] * spl


def _prop_blocks(m_hbm, src_hbm, dst_hbm, w_hbm, base_row, nb,
                 sidxb, didxb, wb, norm_v, rows0, rows1, dinv_v, out_sh,
                 semi, sem0, sem1):
    def idx_issue(b, p):
        base = base_row + b * _BLKC
        return [
            pltpu.async_copy(src_hbm.at[pl.ds(base, _BLKC)], sidxb[p], semi),
            pltpu.async_copy(dst_hbm.at[pl.ds(base, _BLKC)], didxb[p], semi),
            pltpu.async_copy(w_hbm.at[pl.ds(base, _BLKC)], wb[p], semi),
        ]

    def idx_drain(p):
        pltpu.make_async_copy(src_hbm.at[pl.ds(0, _BLKC)], sidxb[p], semi).wait()
        pltpu.make_async_copy(dst_hbm.at[pl.ds(0, _BLKC)], didxb[p], semi).wait()
        pltpu.make_async_copy(w_hbm.at[pl.ds(0, _BLKC)], wb[p], semi).wait()

    def gather_issue(p, j, buf, sem):
        pltpu.async_copy(m_hbm.at[sidxb[p].at[j]], buf, sem)

    def process(p, j, buf, sem):
        for kk in range(_CHUNK // 16):
            sl = pl.ds(kk * 16, 16)
            ds_ = plsc.load_gather(dinv_v, [sidxb[p][j, sl]])
            dd_ = plsc.load_gather(dinv_v, [didxb[p][j, sl]])
            norm_v[sl] = ds_ * wb[p][j, sl] * dd_
        pltpu.make_async_copy(m_hbm.at[pl.ds(0, _CHUNK)], buf, sem).wait()
        _scale_rows(buf, norm_v, _DH)
        pltpu.sync_copy(buf, out_sh.at[didxb[p].at[j]], add=True)

    for h in idx_issue(0, 0):
        h.wait()

    def block(b, p):
        @pl.when(b < nb - 1)
        def _():
            idx_issue(b + 1, 1 - p)

        gather_issue(p, 0, rows0, sem0)

        @pl.loop(0, _BLKC // 2)
        def _(q):
            j0 = 2 * q
            gather_issue(p, j0 + 1, rows1, sem1)
            process(p, j0, rows0, sem0)

            @pl.when(q < _BLKC // 2 - 1)
            def _():
                gather_issue(p, j0 + 2, rows0, sem0)

            process(p, j0 + 1, rows1, sem1)

        @pl.when(b < nb - 1)
        def _():
            idx_drain(1 - p)

    @pl.loop(0, nb // 2)
    def _(bb):
        block(2 * bb, 0)
        block(2 * bb + 1, 1)

    @pl.when(lax.rem(nb, 2) == 1)
    def _():
        block(nb - 1, 0)


def _sc_layer1(m1, src80, dst80, w80):
    @functools.partial(
        pl.kernel,
        out_type=[
            jax.ShapeDtypeStruct((_NC, _NPAD, _DH), jnp.float32),
            jax.ShapeDtypeStruct((_NPAD,), jnp.float32),
        ],
        mesh=_mesh,
        compiler_params=_sc_params,
        scratch_types=[
            pltpu.VMEM((_BLKC, _CHUNK), jnp.int32),
            pltpu.VMEM((_BLKC, _CHUNK), jnp.int32),
            pltpu.VMEM((_BLKC, _CHUNK), jnp.int32),
            pltpu.VMEM((_BLKC, _CHUNK), jnp.int32),
            pltpu.VMEM((_BLKC, _CHUNK), jnp.float32),
            pltpu.VMEM((_BLKC, _CHUNK), jnp.float32),
            pltpu.VMEM((_CHUNK,), jnp.float32),
            pltpu.VMEM((_CHUNK, _DH), jnp.float32),
            pltpu.VMEM((_CHUNK, _DH), jnp.float32),
            pltpu.VMEM((_NPAD,), jnp.float32),
            pltpu.VMEM((_DINV_T,), jnp.float32),
            pltpu.VMEM_SHARED((_NPAD,), jnp.float32),
            pltpu.VMEM_SHARED((_NPAD,), jnp.float32),
            pltpu.VMEM_SHARED((_NPAD, _DH), jnp.float32),
            pltpu.SemaphoreType.DMA,
            pltpu.SemaphoreType.DMA,
            pltpu.SemaphoreType.DMA,
            pltpu.SemaphoreType.DMA,
        ],
    )
    def k(m1_hbm, src_hbm, dst_hbm, w_hbm, p1_hbm, dinv_hbm,
          sidxb0, sidxb1, didxb0, didxb1, wb0, wb1, norm_v, rows0, rows1,
          dinv_v, tmp_v, deg_sh, dinv_sh, out_sh, semi, sem0, sem1, semd):
        cid = lax.axis_index("c")
        sid = lax.axis_index("s")
        sidxb, didxb, wb = [sidxb0, sidxb1], [didxb0, didxb1], [wb0, wb1]

        @pl.loop(0, _CHUNK)
        def _(r):
            for j in range(_DH // 16):
                rows0[r, pl.ds(16 * j, 16)] = jnp.zeros((16,), jnp.float32)

        @pl.loop(0, _DINV_T // 16)
        def _(i):
            _zero16(tmp_v, i * 16)

        pltpu.sync_copy(tmp_v, deg_sh.at[pl.ds(sid * _DINV_T, _DINV_T)])
        for kk in range(_ROWS_T // _CHUNK):
            pltpu.sync_copy(
                rows0, out_sh.at[pl.ds(sid * _ROWS_T + _CHUNK * kk, _CHUNK)])
        plsc.subcore_barrier()

        nbd = _DCHUNK // _BLKC

        def deg_issue(g, p):
            base = sid * _DCHUNK + g * _BLKC
            return [
                pltpu.async_copy(dst_hbm.at[pl.ds(base, _BLKC)], didxb[p],
                                 semi),
                pltpu.async_copy(w_hbm.at[pl.ds(base, _BLKC)], wb[p], semi),
            ]

        def deg_drain(p):
            pltpu.make_async_copy(dst_hbm.at[pl.ds(0, _BLKC)], didxb[p],
                                  semi).wait()
            pltpu.make_async_copy(w_hbm.at[pl.ds(0, _BLKC)], wb[p],
                                  semi).wait()

        for h in deg_issue(0, 0):
            h.wait()

        def deg_block(g, p):
            @pl.when(g < nbd - 1)
            def _():
                deg_issue(g + 1, 1 - p)

            hs = []
            for j in range(_BLKC):
                hs.append(pltpu.async_copy(
                    wb[p].at[j], deg_sh.at[didxb[p].at[j]], semd, add=True))
            for h in hs:
                h.wait()

            @pl.when(g < nbd - 1)
            def _():
                deg_drain(1 - p)

        @pl.loop(0, nbd // 2)
        def _(gg):
            deg_block(2 * gg, 0)
            deg_block(2 * gg + 1, 1)

        pl.delay(10000)
        plsc.subcore_barrier()

        nbase = sid * _DINV_T
        pltpu.sync_copy(deg_sh.at[pl.ds(nbase, _DINV_T)], tmp_v)

        @pl.loop(0, _DINV_T // 16)
        def _(i):
            d = tmp_v[pl.ds(i * 16, 16)] + jnp.full((16,), 1.0, jnp.float32)
            tmp_v[pl.ds(i * 16, 16)] = _newton_rsqrt(d)

        pltpu.sync_copy(tmp_v, dinv_sh.at[pl.ds(nbase, _DINV_T)])

        @pl.when(cid == 0)
        def _():
            pltpu.sync_copy(tmp_v, dinv_hbm.at[pl.ds(nbase, _DINV_T)])

        plsc.subcore_barrier()

        pltpu.sync_copy(dinv_sh, dinv_v)
        base_row = jnp.where(cid == 0, sid * _L1_NCH0,
                             _NS * _L1_NCH0 + sid * _L1_NCH1)
        nb = jnp.where(cid == 0, _L1_NCH0 // _BLKC, _L1_NCH1 // _BLKC)
        _prop_blocks(m1_hbm, src_hbm, dst_hbm, w_hbm, base_row, nb,
                     sidxb, didxb, wb, norm_v, rows0, rows1, dinv_v,
                     out_sh, semi, sem0, sem1)
        plsc.subcore_barrier()

        pltpu.sync_copy(out_sh.at[pl.ds(sid * _ROWS_T, _ROWS_T)],
                        p1_hbm.at[cid, pl.ds(sid * _ROWS_T, _ROWS_T)])

    return k(m1, src80, dst80, w80)


def _sc_layer2(m2, dinv_full, src80, dst80, w80):
    @functools.partial(
        pl.kernel,
        out_type=jax.ShapeDtypeStruct((_NC, _NPAD, _DH), jnp.float32),
        mesh=_mesh,
        compiler_params=_sc_params,
        scratch_types=[
            pltpu.VMEM((_BLKC, _CHUNK), jnp.int32),
            pltpu.VMEM((_BLKC, _CHUNK), jnp.int32),
            pltpu.VMEM((_BLKC, _CHUNK), jnp.int32),
            pltpu.VMEM((_BLKC, _CHUNK), jnp.int32),
            pltpu.VMEM((_BLKC, _CHUNK), jnp.float32),
            pltpu.VMEM((_BLKC, _CHUNK), jnp.float32),
            pltpu.VMEM((_CHUNK,), jnp.float32),
            pltpu.VMEM((_CHUNK, _DH), jnp.float32),
            pltpu.VMEM((_CHUNK, _DH), jnp.float32),
            pltpu.VMEM((_NPAD,), jnp.float32),
            pltpu.VMEM_SHARED((_NPAD, _DH), jnp.float32),
            pltpu.SemaphoreType.DMA,
            pltpu.SemaphoreType.DMA,
            pltpu.SemaphoreType.DMA,
        ],
    )
    def k(m2_hbm, dinv_hbm, src_hbm, dst_hbm, w_hbm, p2_hbm,
          sidxb0, sidxb1, didxb0, didxb1, wb0, wb1, norm_v, rows0, rows1,
          dinv_v, out_sh, semi, sem0, sem1):
        cid = lax.axis_index("c")
        sid = lax.axis_index("s")
        sidxb, didxb, wb = [sidxb0, sidxb1], [didxb0, didxb1], [wb0, wb1]

        hdinv = pltpu.async_copy(dinv_hbm, dinv_v, sem0)

        @pl.loop(0, _CHUNK)
        def _(r):
            for j in range(_DH // 16):
                rows0[r, pl.ds(16 * j, 16)] = jnp.zeros((16,), jnp.float32)

        for kk in range(_ROWS_T // _CHUNK):
            pltpu.sync_copy(
                rows0, out_sh.at[pl.ds(sid * _ROWS_T + _CHUNK * kk, _CHUNK)])
        hdinv.wait()
        plsc.subcore_barrier()

        base_row = jnp.where(cid == 0, sid * _L2_NCH0,
                             _NS * _L2_NCH0 + sid * _L2_NCH1)
        nb = jnp.where(cid == 0, _L2_NCH0 // _BLKC, _L2_NCH1 // _BLKC)
        _prop_blocks(m2_hbm, src_hbm, dst_hbm, w_hbm, base_row, nb,
                     sidxb, didxb, wb, norm_v, rows0, rows1, dinv_v,
                     out_sh, semi, sem0, sem1)
        plsc.subcore_barrier()

        pltpu.sync_copy(out_sh.at[pl.ds(sid * _ROWS_T, _ROWS_T)],
                        p2_hbm.at[cid, pl.ds(sid * _ROWS_T, _ROWS_T)])

    return k(m2, dinv_full, src80, dst80, w80)


_BLK = 1000


def _mm1_body(x_ref, w_ref, o_ref):
    o_ref[...] = jnp.dot(x_ref[...], w_ref[...],
                         preferred_element_type=jnp.float32)


def _mm1(x, W1):
    return pl.pallas_call(
        _mm1_body,
        grid=(_N // _BLK,),
        in_specs=[
            pl.BlockSpec((_BLK, _DIN), lambda i: (i, 0)),
            pl.BlockSpec((_DIN, _DH), lambda i: (0, 0)),
        ],
        out_specs=pl.BlockSpec((_BLK, _DH), lambda i: (i, 0)),
        out_shape=jax.ShapeDtypeStruct((_N, _DH), jnp.float32),
    )(x, W1)


def _mid_body(p1a, p1b, m1, dinv, b1, lw, lb, o_ref):
    dv = dinv[...]
    t = p1a[...] + p1b[...] + dv * dv * m1[...] + b1[...]
    t = jnp.maximum(t, 0.0)
    mu = jnp.mean(t, axis=-1, keepdims=True)
    var = jnp.mean((t - mu) ** 2, axis=-1, keepdims=True)
    o_ref[...] = (t - mu) * lax.rsqrt(var + 1e-5) * lw[...] + lb[...]


def _mid(p1a, p1b, m1, dinv, b1, lw, lb):
    row = lambda i: (i, 0)
    fixed = lambda i: (0, 0)
    return pl.pallas_call(
        _mid_body,
        grid=(_N // _BLK,),
        in_specs=[
            pl.BlockSpec((_BLK, _DH), row),
            pl.BlockSpec((_BLK, _DH), row),
            pl.BlockSpec((_BLK, _DH), row),
            pl.BlockSpec((_BLK, 1), row),
            pl.BlockSpec((1, _DH), fixed),
            pl.BlockSpec((1, _DH), fixed),
            pl.BlockSpec((1, _DH), fixed),
        ],
        out_specs=pl.BlockSpec((_BLK, _DH), row),
        out_shape=jax.ShapeDtypeStruct((_N, _DH), jnp.float32),
    )(p1a, p1b, m1, dinv, b1, lw, lb)


def _fin_body(q2a, q2b, hln, dinv, w2, b2, lw, lb, seg, o_ref, acc):
    i = pl.program_id(0)

    @pl.when(i == 0)
    def _():
        acc[...] = jnp.zeros_like(acc)

    dv = dinv[...]
    t0 = q2a[...] + q2b[...] + dv * dv * hln[...]
    t = jnp.dot(t0, w2[...], preferred_element_type=jnp.float32) + b2[...]
    t = jnp.maximum(t, 0.0)
    mu = jnp.mean(t, axis=-1, keepdims=True)
    var = jnp.mean((t - mu) ** 2, axis=-1, keepdims=True)
    t = (t - mu) * lax.rsqrt(var + 1e-5) * lw[...] + lb[...]

    onehot = (seg[...] == lax.broadcasted_iota(jnp.int32, (1, _G), 1)
              ).astype(jnp.float32)
    t_ext = jnp.concatenate(
        [t, jnp.ones((_BLK, 1), jnp.float32)], axis=1)
    acc[...] += lax.dot_general(onehot, t_ext, (((0,), (0,)), ((), ())),
                                preferred_element_type=jnp.float32)

    @pl.when(i == _N // _BLK - 1)
    def _():
        sums = acc[:, :_DOUT]
        cnt = acc[:, _DOUT:_DOUT + 1]
        pooled = sums / jnp.maximum(cnt, 1.0)
        o_ref[...] = 1.0 / (1.0 + jnp.exp(-pooled))


def _fin(q2a, q2b, hln, dinv, W2, b2, lw, lb, seg):
    row = lambda i: (i, 0)
    fixed = lambda i: (0, 0)
    return pl.pallas_call(
        _fin_body,
        grid=(_N // _BLK,),
        in_specs=[
            pl.BlockSpec((_BLK, _DH), row),
            pl.BlockSpec((_BLK, _DH), row),
            pl.BlockSpec((_BLK, _DH), row),
            pl.BlockSpec((_BLK, 1), row),
            pl.BlockSpec((_DH, _DOUT), fixed),
            pl.BlockSpec((1, _DOUT), fixed),
            pl.BlockSpec((1, _DOUT), fixed),
            pl.BlockSpec((1, _DOUT), fixed),
            pl.BlockSpec((_BLK, 1), row),
        ],
        out_specs=pl.BlockSpec((_G, _G), fixed),
        out_shape=jax.ShapeDtypeStruct((_G, _G), jnp.float32),
        scratch_shapes=[pltpu.VMEM((_G, _DOUT + 1), jnp.float32)],
    )(q2a, q2b, hln, dinv, W2, b2, lw, lb, seg)


def kernel(x, edge_index, edge_weight, data, W1, b1, ln1_w, ln1_b,
           W2, b2, ln2_w, ln2_b):
    epad = _EPAD - _E
    src = jnp.concatenate(
        [edge_index[0], jnp.zeros((epad,), jnp.int32)]
    ).reshape(_EPAD // _CHUNK, _CHUNK)
    dst = jnp.concatenate(
        [edge_index[1], jnp.zeros((epad,), jnp.int32)]
    ).reshape(_EPAD // _CHUNK, _CHUNK)
    ew = jnp.concatenate(
        [edge_weight, jnp.zeros((epad,), jnp.float32)]
    ).reshape(_EPAD // _CHUNK, _CHUNK)

    m1 = _mm1(x, W1)
    p1, dinv_full = _sc_layer1(m1, src, dst, ew)
    dinv = dinv_full[:_N].reshape(_N, 1)

    hln = _mid(p1[0, :_N], p1[1, :_N], m1, dinv,
               b1.reshape(1, _DH), ln1_w.reshape(1, _DH),
               ln1_b.reshape(1, _DH))

    p2 = _sc_layer2(hln, dinv_full, src, dst, ew)

    return _fin(p2[0, :_N], p2[1, :_N], hln, dinv, W2,
                b2.reshape(1, _DOUT), ln2_w.reshape(1, _DOUT),
                ln2_b.reshape(1, _DOUT), data.reshape(_N, 1))
